# Initial kernel scaffold; baseline (speedup 1.0000x reference)
import jax
import jax.numpy as jnp
from jax.experimental import pallas as pl


def kernel(x, edge_index, W1, b1, gamma, beta, running_mean, running_var, W2, b2, eps):
    raise NotImplementedError("write your pallas kernel here")


# speedup vs baseline: 5.8584x; 5.8584x over previous
import functools

import jax
import jax.numpy as jnp
from jax import lax
from jax.experimental import pallas as pl
from jax.experimental.pallas import tpu as pltpu
from jax.experimental.pallas import tpu_sc as plsc

NC, NS = 2, 16
CH = 128


def _sc_scatter(x, src, dst):
    N, D = x.shape
    E = src.shape[0]
    W = NC * NS
    ep_w = E // W
    n_full = ep_w // CH
    tail = ep_w - n_full * CH
    rpw = (N // NS) // 8 * 8
    r_last_extra = N - NS * rpw

    mesh = plsc.VectorSubcoreMesh(core_axis_name="c", subcore_axis_name="s")

    scratch = [
        pltpu.VMEM((CH,), jnp.int32),
        pltpu.VMEM((CH,), jnp.int32),
        pltpu.VMEM((CH, D), jnp.float32),
        pltpu.VMEM_SHARED((N, D), jnp.float32),
        pltpu.SemaphoreType.DMA,
    ]
    if tail:
        scratch += [
            pltpu.VMEM((tail,), jnp.int32),
            pltpu.VMEM((tail,), jnp.int32),
            pltpu.VMEM((tail, D), jnp.float32),
        ]

    @functools.partial(
        pl.kernel,
        mesh=mesh,
        out_type=jax.ShapeDtypeStruct((NC, N, D), jnp.float32),
        scratch_types=scratch,
    )
    def k(x_hbm, src_hbm, dst_hbm, out_hbm, src_v, dst_v, rows_v, agg_sh,
          sem, *tail_bufs):
        c = lax.axis_index("c")
        s = lax.axis_index("s")
        w = c * NS + s

        @pl.loop(0, CH)
        def _(i):
            for j in range(D // 16):
                rows_v.at[pl.ds(i, 1), pl.ds(j * 16, 16)][...] = (
                    jnp.zeros((1, 16), jnp.float32))

        row0 = pl.multiple_of(s * rpw, 8)

        def copy_rows(nrows, dst_ref):
            o = 0
            while o < nrows:
                n = min(CH, nrows - o)
                pltpu.sync_copy(rows_v.at[pl.ds(0, n)] if n < CH else rows_v,
                                dst_ref.at[pl.ds(pl.multiple_of(row0 + o, 8),
                                                 n)])
                o += n

        copy_rows(rpw, agg_sh)

        @pl.when(s == NS - 1)
        def _():
            pltpu.sync_copy(
                rows_v.at[pl.ds(0, r_last_extra)],
                agg_sh.at[pl.ds(pl.multiple_of(row0 + rpw, 8), r_last_extra)])
        plsc.subcore_barrier()

        base = pl.multiple_of(w * ep_w, 8)

        @pl.loop(0, n_full)
        def _(kk):
            off = pl.multiple_of(base + kk * CH, 8)
            pltpu.sync_copy(src_hbm.at[pl.ds(off, CH)], src_v)
            pltpu.sync_copy(dst_hbm.at[pl.ds(off, CH)], dst_v)
            pltpu.async_copy(x_hbm.at[src_v], rows_v, sem).wait()
            pltpu.sync_copy(rows_v, agg_sh.at[dst_v], add=True)

        if tail:
            src_t, dst_t, rows_t = tail_bufs
            off = pl.multiple_of(base + n_full * CH, 8)
            pltpu.sync_copy(src_hbm.at[pl.ds(off, tail)], src_t)
            pltpu.sync_copy(dst_hbm.at[pl.ds(off, tail)], dst_t)
            pltpu.async_copy(x_hbm.at[src_t], rows_t, sem).wait()
            pltpu.sync_copy(rows_t, agg_sh.at[dst_t], add=True)

        plsc.subcore_barrier()
        pltpu.sync_copy(agg_sh.at[pl.ds(row0, rpw)],
                        out_hbm.at[c].at[pl.ds(row0, rpw)])

        @pl.when(s == NS - 1)
        def _():
            off2 = pl.multiple_of(row0 + rpw, 8)
            pltpu.sync_copy(agg_sh.at[pl.ds(off2, r_last_extra)],
                            out_hbm.at[c].at[pl.ds(off2, r_last_extra)])

    return k(x, src, dst)


def _mlp_body(x_ref, agg_ref, w1_ref, b1_ref, g_ref, be_ref, mu_ref,
              var_ref, w2_ref, b2_ref, eps_ref, o_ref):
    eps = eps_ref[0, 0]
    h = (1.0 + eps) * x_ref[...] + agg_ref[0] + agg_ref[1]
    h = lax.dot_general(h, w1_ref[...], (((1,), (1,)), ((), ())),
                        preferred_element_type=jnp.float32,
                        precision=lax.Precision.HIGHEST)
    h = h + b1_ref[...]
    scale = g_ref[...] * lax.rsqrt(var_ref[...] + 1e-5)
    h = (h - mu_ref[...]) * scale + be_ref[...]
    h = jnp.maximum(h, 0.0)
    h = lax.dot_general(h, w2_ref[...], (((1,), (1,)), ((), ())),
                        preferred_element_type=jnp.float32,
                        precision=lax.Precision.HIGHEST)
    o_ref[...] = h + b2_ref[...]


def kernel(x, edge_index, W1, b1, gamma, beta, running_mean, running_var,
           W2, b2, eps):
    N, D = x.shape
    src = edge_index[0]
    dst = edge_index[1]

    agg2 = _sc_scatter(x, src, dst)

    R = 400
    vec = lambda v: v.reshape(1, D)
    full = lambda shp: pl.BlockSpec(shp, lambda i: tuple(0 for _ in shp))
    out = pl.pallas_call(
        _mlp_body,
        grid=(N // R,),
        in_specs=[
            pl.BlockSpec((R, D), lambda i: (i, 0)),
            pl.BlockSpec((NC, R, D), lambda i: (0, i, 0)),
            full((D, D)),
            full((1, D)),
            full((1, D)),
            full((1, D)),
            full((1, D)),
            full((1, D)),
            full((D, D)),
            full((1, D)),
            pl.BlockSpec(memory_space=pltpu.SMEM),
        ],
        out_specs=pl.BlockSpec((R, D), lambda i: (i, 0)),
        out_shape=jax.ShapeDtypeStruct((N, D), jnp.float32),
    )(x, agg2, W1, vec(b1), vec(gamma), vec(beta), vec(running_mean),
      vec(running_var), W2, vec(b2), eps.reshape(1, 1))
    return out
